# baseline (speedup 1.0000x reference)
import functools

import numpy as np
import jax
import jax.numpy as jnp
from jax import lax
from jax.experimental import pallas as pl
from jax.experimental.pallas import tpu as pltpu
from jax.experimental.pallas import tpu_sc as plsc

_SW = 8
_N_NODE = 64
_N_STAGES = 4
_N_OPS = 8192
_N_NODES_T = _N_STAGES * _N_NODE
_N_ROWS = _N_NODES_T * _SW

_SC_NODES = 64
_TC_NODES = _N_NODES_T - _SC_NODES
_TC_BLK = 16


def _pairs():
    I, J = [], []
    for i in range(2, _N_NODE + 1):
        for j in range(i + 1, _N_NODE + 2):
            I.append(i)
            J.append(j)
    return (np.asarray(I, np.int32).reshape(-1, 1),
            np.asarray(J, np.int32).reshape(-1, 1))


_I_IDX, _J_IDX = _pairs()


_NC, _NS = 2, 16
_NW = _NC * _NS
_SC_ROWS = _SC_NODES * _SW
_ROWS_PER_W = _SC_ROWS // _NW
_CHUNK_R = 4
_N_CHUNKS = _ROWS_PER_W // _CHUNK_R


@functools.cache
def _sc_rowmax_fn():
    return functools.partial(
        pl.kernel,
        mesh=plsc.VectorSubcoreMesh(core_axis_name="c", subcore_axis_name="s"),
        out_type=jax.ShapeDtypeStruct((_SC_ROWS, 16), jnp.float32),
        scratch_types=[
            pltpu.VMEM((_CHUNK_R, _N_OPS), jnp.float32),
            pltpu.VMEM((_CHUNK_R, _N_OPS), jnp.float32),
            pltpu.VMEM((_ROWS_PER_W, 16), jnp.float32),
            pltpu.SemaphoreType.DMA,
            pltpu.SemaphoreType.DMA,
        ],
    )(_sc_rowmax_body)


def _sc_rowmax_body(a_hbm, out_hbm, buf0, buf1, ovec, sem0, sem1):
    wid = lax.axis_index("s") * _NC + lax.axis_index("c")
    base = wid * _ROWS_PER_W
    bufs = (buf0, buf1)
    sems = (sem0, sem1)

    def copy_in(c, buf, sem):
        return pltpu.make_async_copy(
            a_hbm.at[pl.ds(base + c * _CHUNK_R, _CHUNK_R)], buf, sem)

    copy_in(0, buf0, sem0).start()
    lane = lax.iota(jnp.int32, 16)
    neg_inf = jnp.full((16,), -jnp.inf, jnp.float32)
    for c in range(_N_CHUNKS):
        buf, sem = bufs[c % 2], sems[c % 2]
        if c + 1 < _N_CHUNKS:
            copy_in(c + 1, bufs[(c + 1) % 2], sems[(c + 1) % 2]).start()
        copy_in(c, buf, sem).wait()
        for r in range(_CHUNK_R):
            row = c * _CHUNK_R + r
            last = buf[r, pl.ds(_N_OPS - 16, 16)]
            buf[r, pl.ds(_N_OPS - 16, 16)] = jnp.where(lane < 15, last,
                                                       -jnp.inf)

            def body(i, acc, _r=r):
                b = i * 128
                m0 = jnp.maximum(buf[_r, pl.ds(b, 16)],
                                 buf[_r, pl.ds(b + 16, 16)])
                m1 = jnp.maximum(buf[_r, pl.ds(b + 32, 16)],
                                 buf[_r, pl.ds(b + 48, 16)])
                m2 = jnp.maximum(buf[_r, pl.ds(b + 64, 16)],
                                 buf[_r, pl.ds(b + 80, 16)])
                m3 = jnp.maximum(buf[_r, pl.ds(b + 96, 16)],
                                 buf[_r, pl.ds(b + 112, 16)])
                return jnp.maximum(
                    acc, jnp.maximum(jnp.maximum(m0, m1), jnp.maximum(m2, m3)))

            acc = lax.fori_loop(0, _N_OPS // 128, body, neg_inf)
            ovec[row, :] = acc
    pltpu.sync_copy(ovec, out_hbm.at[pl.ds(base, _ROWS_PER_W)])


def _tc_rowmax_body(a_ref, o_ref):
    x = a_ref[...]
    lane = jax.lax.broadcasted_iota(jnp.int32, x.shape, 2)
    x = jnp.where(lane < _N_OPS - 1, x, -jnp.inf)
    o_ref[...] = jnp.max(x, axis=2)


def _tc_rowmax(alpha3):
    return pl.pallas_call(
        _tc_rowmax_body,
        grid=(_TC_NODES // _TC_BLK,),
        in_specs=[pl.BlockSpec((_TC_BLK, _SW, _N_OPS),
                               lambda i: (i + _SC_NODES // _TC_BLK, 0, 0))],
        out_specs=pl.BlockSpec((_TC_BLK, _SW), lambda i: (i, 0)),
        out_shape=jax.ShapeDtypeStruct((_TC_NODES, _SW), jnp.float32),
    )(alpha3)


def _epilogue_body(sc_ref, tc_ref, beta_ref, i_ref, j_ref, theta_ref,
                   out_ref):
    em_sc = jnp.max(sc_ref[...], axis=2)
    em = jnp.concatenate([em_sc, tc_ref[...]], axis=0)
    m = jnp.max(em, axis=1, keepdims=True)
    p = jnp.exp(em - m)
    e = p / jnp.sum(p, axis=1, keepdims=True)

    ED = jnp.zeros((_N_STAGES, 128), jnp.float32)
    lane = jax.lax.broadcasted_iota(jnp.int32, (_N_STAGES, 128), 1)
    for j in range(2, _N_NODE + 2):
        rows = jnp.concatenate(
            [e[s * _N_NODE + j - 2][None, :] for s in range(_N_STAGES)], axis=0
        )
        if j < _SW:
            contrib = jnp.sum(rows[:, :j] * (ED[:, :j] + 1.0), axis=1,
                              keepdims=True)
        else:
            contrib = jnp.sum(rows * (ED[:, j - _SW:j] + 1.0), axis=1,
                              keepdims=True)
        ED = jnp.where(lane == j, ED + contrib, ED)

    beta = beta_ref[...]
    bm = jnp.max(beta, axis=1, keepdims=True)
    be = jnp.exp(beta - bm)
    denom = jnp.sum(be, axis=1, keepdims=True)
    n_iota = jax.lax.broadcasted_iota(jnp.int32, (_I_IDX.shape[0], 128), 1)
    oh = ((i_ref[...] == n_iota).astype(jnp.float32) +
          (j_ref[...] == n_iota).astype(jnp.float32))
    T = jax.lax.dot_general(be, oh, (((1,), (0,)), ((), ())),
                            preferred_element_type=jnp.float32)
    depth = jnp.sum(T * ED, axis=1, keepdims=True) / denom
    out_ref[...] = jnp.sum(theta_ref[...] * depth, axis=0, keepdims=True)


def _epilogue(em_sc16, em_tc, beta, theta):
    return pl.pallas_call(
        _epilogue_body,
        out_shape=jax.ShapeDtypeStruct((1, 1), jnp.float32),
    )(em_sc16, em_tc, beta, jnp.asarray(_I_IDX), jnp.asarray(_J_IDX),
      theta.reshape(_N_STAGES, 1))


def kernel(alpha, beta, theta):
    a2 = alpha.reshape(_N_ROWS, _N_OPS)
    em_sc16 = _sc_rowmax_fn()(a2).reshape(_SC_NODES, _SW, 16)
    em_tc = _tc_rowmax(alpha)
    return _epilogue(em_sc16, em_tc, beta, theta)[0, 0]
